# baseline (speedup 1.0000x reference)
import functools

import jax
import jax.numpy as jnp
from jax import lax
from jax.experimental import pallas as pl
from jax.experimental.pallas import tpu as pltpu
from jax.experimental.pallas import tpu_sc as plsc

_NC = 2
_NS = 16
_L = 16


@functools.lru_cache(maxsize=None)
def _make_sc_diag_scatter(n, diag_len):
    off = n - diag_len
    nw = _NC * _NS
    rw = n // nw
    cw = -(-(rw + off) // 128) * 128

    mesh = plsc.VectorSubcoreMesh(
        core_axis_name="c", subcore_axis_name="s",
        num_cores=_NC, num_subcores=_NS,
    )

    def _patch(w_v, s_v, doff, nrows):
        for r in range(nrows):
            ct = doff + r
            a = (ct // _L) * _L
            l = ct - a
            val = s_v[pl.ds(_L + r - l, _L)]
            mask = lax.iota(jnp.int32, _L) == l
            cur = w_v[r, pl.ds(a, _L)]
            w_v[r, pl.ds(a, _L)] = jnp.where(mask, val, cur)

    @functools.partial(
        pl.kernel,
        out_type=jax.ShapeDtypeStruct((n, n), jnp.float32),
        mesh=mesh,
        scratch_types=[
            pltpu.VMEM((rw, cw), jnp.float32),
            pltpu.VMEM((rw + 2 * _L,), jnp.float32),
        ],
    )
    def sc_kernel(x_hbm, src_hbm, out_hbm, w_v, s_v):
        wid = lax.axis_index("s") * _NC + lax.axis_index("c")
        base = wid * rw
        pltpu.sync_copy(x_hbm.at[pl.ds(base, rw)], out_hbm.at[pl.ds(base, rw)])
        col_start = pl.multiple_of(jnp.minimum(base, n - cw), 128)
        pltpu.sync_copy(x_hbm.at[pl.ds(base, rw), pl.ds(col_start, cw)], w_v)
        pltpu.sync_copy(src_hbm.at[pl.ds(base, rw)], s_v.at[pl.ds(_L, rw)])

        last_base = (nw - 1) * rw
        last_cs = min(last_base, n - cw)

        @pl.when(wid != nw - 1)
        def _():
            _patch(w_v, s_v, off, rw)

        @pl.when(wid == nw - 1)
        def _():
            _patch(w_v, s_v, last_base + off - last_cs, diag_len - last_base)

        pltpu.sync_copy(w_v, out_hbm.at[pl.ds(base, rw), pl.ds(col_start, cw)])

    return sc_kernel


def kernel(x, src, offset, dim1, dim2):
    n = x.shape[0]
    diag_len = src.shape[0]
    src_pad = jnp.pad(src, (0, n - diag_len))
    return _make_sc_diag_scatter(n, diag_len)(x, src_pad)


# speedup vs baseline: 1.0042x; 1.0042x over previous
import functools

import jax
import jax.numpy as jnp
from jax.experimental import pallas as pl
from jax.experimental.pallas import tpu as pltpu

_K = 16


def _body(n, rs):
    def body(x_ref, o_ref, sems):
        for k in range(_K):
            pltpu.make_async_copy(
                x_ref.at[pl.ds(k * rs, rs)],
                o_ref.at[pl.ds(k * rs, rs)],
                sems.at[k],
            ).start()
        for k in range(_K):
            pltpu.make_async_copy(
                x_ref.at[pl.ds(k * rs, rs)],
                o_ref.at[pl.ds(k * rs, rs)],
                sems.at[k],
            ).wait()
    return body


def kernel(x, src, offset, dim1, dim2):
    n = x.shape[0]
    rs = n // _K
    return pl.pallas_call(
        _body(n, rs),
        out_shape=jax.ShapeDtypeStruct((n, n), x.dtype),
        in_specs=[pl.BlockSpec(memory_space=pltpu.HBM)],
        out_specs=pl.BlockSpec(memory_space=pltpu.HBM),
        scratch_shapes=[pltpu.SemaphoreType.DMA((_K,))],
    )(x)


# speedup vs baseline: 38.2550x; 38.0932x over previous
import functools

import jax
import jax.numpy as jnp
from jax import lax
from jax.experimental import pallas as pl
from jax.experimental.pallas import tpu as pltpu
from jax.experimental.pallas import tpu_sc as plsc

_NC = 2
_NS = 16
_L = 16
_CR = 8
_CC = 4096


@functools.lru_cache(maxsize=None)
def _make_sc_diag_scatter(n, diag_len):
    off = n - diag_len
    nw = _NC * _NS
    rw = n // nw
    ncc = n // _CC
    nchunks = (rw // _CR) * ncc

    mesh = plsc.VectorSubcoreMesh(
        core_axis_name="c", subcore_axis_name="s",
        num_cores=_NC, num_subcores=_NS,
    )

    @functools.partial(
        pl.kernel,
        out_type=jax.ShapeDtypeStruct((n, n), jnp.float32),
        mesh=mesh,
        scratch_types=[
            pltpu.VMEM((2, _CR, _CC), jnp.float32),
            pltpu.VMEM((rw + 2 * _L,), jnp.float32),
            pltpu.SemaphoreType.DMA((2,)),
            pltpu.SemaphoreType.DMA((2,)),
        ],
    )
    def sc_kernel(x_hbm, src_hbm, out_hbm, buf, s_v, in_sem, out_sem):
        wid = lax.axis_index("s") * _NC + lax.axis_index("c")
        base = wid * rw
        pltpu.sync_copy(src_hbm.at[pl.ds(base, rw)], s_v.at[pl.ds(_L, rw)])

        def slab(c):
            k, cc = divmod(c, ncc)
            return (pl.ds(base + k * _CR, _CR), pl.ds(cc * _CC, _CC))

        def chunk_in(c, b, do_wait):
            cp = pltpu.make_async_copy(x_hbm.at[slab(c)], buf.at[b], in_sem.at[b])
            cp.wait() if do_wait else cp.start()

        def chunk_out(c, b, do_wait):
            cp = pltpu.make_async_copy(buf.at[b], out_hbm.at[slab(c)], out_sem.at[b])
            cp.wait() if do_wait else cp.start()

        def patch(c, b):
            k, cc = divmod(c, ncc)
            for i in range(_CR):
                r = k * _CR + i
                g = base + r
                l = (r + off) % _L
                gc = g + off
                a = gc - l - cc * _CC
                cond = (a >= 0) & (a < _CC) & (g < diag_len)
                a_s = pl.multiple_of(jnp.clip(a, 0, _CC - _L), _L)
                val = s_v[pl.ds(_L + r - l, _L)]
                cf = cond.astype(jnp.float32)
                mask = lax.iota(jnp.int32, _L) == l
                cur = buf[b, i, pl.ds(a_s, _L)]
                buf[b, i, pl.ds(a_s, _L)] = jnp.where(
                    mask, cur + cf * (val - cur), cur)

        chunk_in(0, 0, False)
        for c in range(nchunks):
            b = c % 2
            if c + 1 < nchunks:
                if c >= 1:
                    chunk_out(c - 1, 1 - b, True)
                chunk_in(c + 1, 1 - b, False)
            chunk_in(c, b, True)
            patch(c, b)
            chunk_out(c, b, False)
        chunk_out(nchunks - 2, nchunks % 2, True)
        chunk_out(nchunks - 1, (nchunks - 1) % 2, True)

    return sc_kernel


def kernel(x, src, offset, dim1, dim2):
    n = x.shape[0]
    diag_len = src.shape[0]
    src_pad = jnp.pad(src, (0, n - diag_len))
    return _make_sc_diag_scatter(n, diag_len)(x, src_pad)


# speedup vs baseline: 38.4223x; 1.0044x over previous
import functools

import jax
import jax.numpy as jnp
from jax import lax
from jax.experimental import pallas as pl
from jax.experimental.pallas import tpu as pltpu
from jax.experimental.pallas import tpu_sc as plsc

_NC = 2
_NS = 16
_L = 16
_CR = 8
_CC = 4096
_NB = 3


@functools.lru_cache(maxsize=None)
def _make_sc_diag_scatter(n, diag_len):
    off = n - diag_len
    nw = _NC * _NS
    rw = n // nw
    ncc = n // _CC
    nchunks = (rw // _CR) * ncc

    mesh = plsc.VectorSubcoreMesh(
        core_axis_name="c", subcore_axis_name="s",
        num_cores=_NC, num_subcores=_NS,
    )

    @functools.partial(
        pl.kernel,
        out_type=jax.ShapeDtypeStruct((n, n), jnp.float32),
        mesh=mesh,
        scratch_types=[
            pltpu.VMEM((_NB, _CR, _CC), jnp.float32),
            pltpu.VMEM((rw + 2 * _L,), jnp.float32),
            pltpu.SemaphoreType.DMA((_NB,)),
            pltpu.SemaphoreType.DMA((_NB,)),
        ],
    )
    def sc_kernel(x_hbm, src_hbm, out_hbm, buf, s_v, in_sem, out_sem):
        wid = lax.axis_index("s") * _NC + lax.axis_index("c")
        base = wid * rw
        pltpu.sync_copy(src_hbm.at[pl.ds(base, rw)], s_v.at[pl.ds(_L, rw)])

        def slab(c):
            k, cc = divmod(c, ncc)
            return (pl.ds(base + k * _CR, _CR), pl.ds(cc * _CC, _CC))

        def chunk_in(c, b, do_wait):
            cp = pltpu.make_async_copy(x_hbm.at[slab(c)], buf.at[b], in_sem.at[b])
            cp.wait() if do_wait else cp.start()

        def chunk_out(c, b, do_wait):
            cp = pltpu.make_async_copy(buf.at[b], out_hbm.at[slab(c)], out_sem.at[b])
            cp.wait() if do_wait else cp.start()

        def patch(c, b):
            k, cc = divmod(c, ncc)
            for i in range(_CR):
                r = k * _CR + i
                g = base + r
                l = (r + off) % _L
                gc = g + off
                a = gc - l - cc * _CC
                cond = (a >= 0) & (a < _CC) & (g < diag_len)
                a_s = pl.multiple_of(jnp.clip(a, 0, _CC - _L), _L)
                val = s_v[pl.ds(_L + r - l, _L)]
                cf = cond.astype(jnp.float32)
                mask = lax.iota(jnp.int32, _L) == l
                cur = buf[b, i, pl.ds(a_s, _L)]
                buf[b, i, pl.ds(a_s, _L)] = jnp.where(
                    mask, cur + cf * (val - cur), cur)

        chunk_in(0, 0, False)
        chunk_in(1, 1, False)
        for c in range(nchunks):
            b = c % _NB
            p = c + _NB - 1
            if p < nchunks:
                if c >= 1:
                    chunk_out(c - 1, (c - 1) % _NB, True)
                chunk_in(p, p % _NB, False)
            chunk_in(c, b, True)
            patch(c, b)
            chunk_out(c, b, False)
        for t in range(max(0, nchunks - _NB), nchunks):
            chunk_out(t, t % _NB, True)

    return sc_kernel


def kernel(x, src, offset, dim1, dim2):
    n = x.shape[0]
    diag_len = src.shape[0]
    src_pad = jnp.pad(src, (0, n - diag_len))
    return _make_sc_diag_scatter(n, diag_len)(x, src_pad)
